# Initial kernel scaffold; baseline (speedup 1.0000x reference)
import jax
import jax.numpy as jnp
from jax.experimental import pallas as pl


def kernel(logits, targets):
    raise NotImplementedError("write your pallas kernel here")


# speedup vs baseline: 45.4136x; 45.4136x over previous
import functools

import jax
import jax.numpy as jnp
from jax import lax
from jax.experimental import pallas as pl
from jax.experimental.pallas import tpu as pltpu

_DELTA = 5.0
_SCALE = 10.0
_B = 64
_C = 100000
_NEG = int(0.01 * (_C - 1))
_ROWS_PER_BLK = 8
_W = 100096


def _mmcl_body(x_ref, t_ref, out_ref, k_ref):
    nblk = pl.num_programs(0)
    x = x_ref[...]
    t = t_ref[...]
    R = x.shape[0]
    col = lax.broadcasted_iota(jnp.int32, (R, _W), 1)
    is_t = col == t

    bits = lax.bitcast_convert_type(x, jnp.uint32)
    sgn = (bits >> jnp.uint32(31)).astype(jnp.uint32)
    flip = jnp.where(sgn == jnp.uint32(1),
                     jnp.uint32(0xFFFFFFFF), jnp.uint32(0x80000000))
    key = bits ^ flip
    key = jnp.where(is_t, jnp.uint32(0), key)
    k_ref[...] = key

    pos = jnp.sum(jnp.where(is_t, x, 0.0), axis=1, keepdims=True)
    vmax = jnp.max(jnp.where(is_t, -jnp.inf, x), axis=1, keepdims=True)

    def step(i, v):
        bit = (jnp.uint32(31) - i.astype(jnp.uint32))
        cand = v | lax.shift_left(jnp.uint32(1), bit)
        cnt = jnp.sum((k_ref[...] >= cand).astype(jnp.int32),
                      axis=1, keepdims=True)
        return jnp.where(cnt >= _NEG, cand, v)

    v = lax.fori_loop(0, 32, step, jnp.zeros((R, 1), jnp.uint32))

    vb = jnp.where(v >= jnp.uint32(0x80000000),
                   v ^ jnp.uint32(0x80000000), ~v)
    vf = lax.bitcast_convert_type(vb, jnp.float32)

    key = k_ref[...]
    m = jnp.maximum(pos, vmax)
    gt = key > v
    c = jnp.sum(gt.astype(jnp.int32), axis=1, keepdims=True)
    e = jnp.sum(jnp.where(gt, jnp.exp(_SCALE * (x - m)), 0.0),
                axis=1, keepdims=True)
    sumexp = (e
              + (jnp.float32(_NEG) - c.astype(jnp.float32))
              * jnp.exp(_SCALE * (vf - m))
              + jnp.exp(_SCALE * (pos - m)))
    row_loss = _SCALE * m + jnp.log(sumexp) - _SCALE * pos
    blk = jnp.sum(row_loss) * (1.0 / _B)

    prev = jnp.where(pl.program_id(0) == 0, 0.0, out_ref[0, 0])
    out_ref[0, 0] = prev + blk


@jax.jit
def kernel(logits, targets):
    xp = jnp.pad(logits, ((0, 0), (0, _W - _C)),
                 constant_values=-jnp.inf)
    t2 = targets.reshape(_B, 1).astype(jnp.int32)
    grid = _B // _ROWS_PER_BLK
    out = pl.pallas_call(
        _mmcl_body,
        grid=(grid,),
        in_specs=[
            pl.BlockSpec((_ROWS_PER_BLK, _W), lambda i: (i, 0)),
            pl.BlockSpec((_ROWS_PER_BLK, 1), lambda i: (i, 0)),
        ],
        out_specs=pl.BlockSpec(memory_space=pltpu.SMEM),
        out_shape=jax.ShapeDtypeStruct((1, 1), jnp.float32),
        scratch_shapes=[pltpu.VMEM((_ROWS_PER_BLK, _W), jnp.uint32)],
    )(xp, t2)
    return out[0, 0]
